# baseline (speedup 1.0000x reference)
import functools

import jax
import jax.numpy as jnp
from jax import lax
from jax.experimental import pallas as pl
from jax.experimental.pallas import tpu as pltpu
from jax.experimental.pallas import tpu_sc as plsc

_NC, _NS = 2, 16
_NW = _NC * _NS

_H, _W, _D = 1024, 1024, 48
_R = _H * _W
_B = 524288
_BANDS = _D // 8
_WT = _W // 128
_OT = _B // 128

_mesh = plsc.VectorSubcoreMesh(core_axis_name="c", subcore_axis_name="s")
_params = pltpu.CompilerParams(use_tc_tiling_on_sc=False,
                               needs_layout_passes=False)


def _wid():
    return lax.axis_index("s") * _NC + lax.axis_index("c")


@functools.partial(
    pl.kernel,
    out_type=jax.ShapeDtypeStruct((_R * _D,), jnp.float32),
    mesh=_mesh,
    scratch_types=[
        pltpu.VMEM((2, _BANDS, 8, 128), jnp.float32),
        pltpu.VMEM((2, 128 * _D), jnp.float32),
        pltpu.SemaphoreType.DMA,
        pltpu.SemaphoreType.DMA,
        pltpu.SemaphoreType.DMA,
        pltpu.SemaphoreType.DMA,
    ],
    compiler_params=_params,
)
def _transpose_kernel(tab_v, rm_out, src_v, dst_v, si0, si1, so0, so1):
    wid = _wid()
    h0 = wid * (_H // _NW)
    n_blocks = (_H // _NW) * _WT
    sin = (si0, si1)
    sout = (so0, so1)
    i16 = lax.iota(jnp.int32, 16)

    def src_slice(blk):
        h = h0 + blk // _WT
        wt = blk % _WT
        return tab_v.at[h, :, wt]

    def out_slice(blk):
        return rm_out.at[pl.ds((h0 * _WT + blk) * 128 * _D, 128 * _D)]

    for u in (0, 1):
        pltpu.async_copy(src_slice(u), src_v.at[u], sin[u])

    def body(g, carry):
        for u in (0, 1):
            i = 2 * g + u
            pltpu.make_async_copy(src_slice(i), src_v.at[u], sin[u]).wait()

            @pl.when(g > 0)
            def _():
                pltpu.make_async_copy(dst_v.at[u], out_slice(i - 2),
                                      sout[u]).wait()

            @plsc.parallel_loop(0, 48, unroll=4)
            def shuf(q):
                t = q // _BANDS
                bb = q % _BANDS
                row48 = (t * 16 + i16) * _D + bb * 8
                for ss in range(8):
                    vals = src_v[u, bb, ss, pl.ds(t * 16, 16)]
                    plsc.store_scatter(dst_v.at[u], [row48 + ss], vals)

            pltpu.async_copy(dst_v.at[u], out_slice(i), sout[u])

            @pl.when(g <= (n_blocks // 2 - 2))
            def _():
                pltpu.async_copy(src_slice(i + 2), src_v.at[u], sin[u])
        return carry

    lax.fori_loop(0, n_blocks // 2, body, 0)
    for u in (0, 1):
        i = n_blocks - 2 + u
        pltpu.make_async_copy(dst_v.at[u], out_slice(i), sout[u]).wait()


@functools.partial(
    pl.kernel,
    out_type=jax.ShapeDtypeStruct((_BANDS * _OT * 1024,), jnp.float32),
    mesh=_mesh,
    scratch_types=[
        pltpu.VMEM((_B // _NW,), jnp.int32),
        pltpu.VMEM((2, 512, _D), jnp.float32),
        pltpu.VMEM((2, 4 * _BANDS * 1024), jnp.float32),
        pltpu.SemaphoreType.DMA,
        pltpu.SemaphoreType.DMA,
        pltpu.SemaphoreType.DMA,
        pltpu.SemaphoreType.DMA,
    ],
    compiler_params=_params,
)
def _gather_kernel(rm_tab, idx_hbm, out_v, idx_v, rows_v, dst_v,
                   sg0, sg1, so0, so1):
    wid = _wid()
    b_per_w = _B // _NW
    n_blocks = b_per_w // 512
    ot0 = wid * (b_per_w // 128)
    sg = (sg0, sg1)
    so = (so0, so1)
    pltpu.sync_copy(idx_hbm.at[pl.ds(wid * b_per_w, b_per_w)], idx_v)
    i16 = lax.iota(jnp.int32, 16)
    flat_idx = []
    for p in range(3):
        dv = p * 16 + i16
        flat_idx.append((dv % 8) * 128 + (dv // 8) * 1024)

    def gather(i, u):
        return pltpu.async_copy(
            rm_tab.at[idx_v.at[pl.ds(i * 512, 512)]], rows_v.at[u], sg[u])

    def out_copies(i, u, fn):
        for sb in range(4):
            ot = ot0 + i * 4 + sb
            for bb in range(_BANDS):
                fn(dst_v.at[u, pl.ds(sb * (_BANDS * 1024) + bb * 1024, 1024)],
                   out_v.at[pl.ds(bb * (_OT * 1024) + ot * 1024, 1024)],
                   so[u])

    for u in (0, 1):
        gather(u, u)

    def body(g, carry):
        for u in (0, 1):
            i = 2 * g + u
            pltpu.make_async_copy(
                rm_tab.at[idx_v.at[pl.ds(i * 512, 512)]],
                rows_v.at[u], sg[u]).wait()

            @pl.when(g > 0)
            def _():
                out_copies(i - 2, u,
                           lambda s, d, m: pltpu.make_async_copy(s, d, m).wait())

            @plsc.parallel_loop(0, 512, step=4, unroll=4)
            def shuf(j0):
                sb = j0 // 128
                base = sb * (_BANDS * 1024) - sb * 128
                for jj in range(4):
                    j = j0 + jj
                    for p in range(3):
                        vals = rows_v[u, j, pl.ds(p * 16, 16)]
                        plsc.store_scatter(
                            dst_v.at[u], [flat_idx[p] + (base + j)], vals)

            out_copies(i, u, pltpu.async_copy)

            @pl.when(g <= (n_blocks // 2 - 2))
            def _():
                gather(i + 2, u)
        return carry

    lax.fori_loop(0, n_blocks // 2, body, 0)
    for u in (0, 1):
        i = n_blocks - 2 + u
        out_copies(i, u, lambda s, d, m: pltpu.make_async_copy(s, d, m).wait())


def kernel(feat_plane, mask_indices):
    tab_v = feat_plane.reshape(_H, _WT, 128, _BANDS, 8).transpose(0, 3, 1, 4, 2)
    idx = mask_indices.astype(jnp.int32)
    rm = _transpose_kernel(tab_v).reshape(_R, _D)
    out_flat = _gather_kernel(rm, idx)
    return (out_flat.reshape(_BANDS, _OT, 8, 128)
            .transpose(1, 3, 0, 2)
            .reshape(_B, _D))


# speedup vs baseline: 1.0049x; 1.0049x over previous
import functools

import jax
import jax.numpy as jnp
from jax import lax
from jax.experimental import pallas as pl
from jax.experimental.pallas import tpu as pltpu
from jax.experimental.pallas import tpu_sc as plsc

_NC, _NS = 2, 16
_NW = _NC * _NS

_H, _W, _D = 1024, 1024, 48
_R = _H * _W
_B = 524288
_BANDS = _D // 8
_WT = _W // 128
_OT = _B // 128

_mesh = plsc.VectorSubcoreMesh(core_axis_name="c", subcore_axis_name="s")
_params = pltpu.CompilerParams(use_tc_tiling_on_sc=False,
                               needs_layout_passes=False)


def _wid():
    return lax.axis_index("s") * _NC + lax.axis_index("c")


@functools.partial(
    pl.kernel,
    out_type=jax.ShapeDtypeStruct((_R * _D,), jnp.float32),
    mesh=_mesh,
    scratch_types=[
        pltpu.VMEM((2, _BANDS, 8, 128), jnp.float32),
        pltpu.VMEM((2, 128 * _D), jnp.float32),
        pltpu.SemaphoreType.DMA,
        pltpu.SemaphoreType.DMA,
        pltpu.SemaphoreType.DMA,
        pltpu.SemaphoreType.DMA,
    ],
    compiler_params=_params,
)
def _transpose_kernel(tab_v, rm_out, src_v, dst_v, si0, si1, so0, so1):
    wid = _wid()
    h0 = wid * (_H // _NW)
    n_blocks = (_H // _NW) * _WT
    sin = (si0, si1)
    sout = (so0, so1)
    i16 = lax.iota(jnp.int32, 16)

    def src_slice(blk):
        h = h0 + blk // _WT
        wt = blk % _WT
        return tab_v.at[h, :, wt]

    def out_slice(blk):
        return rm_out.at[pl.ds((h0 * _WT + blk) * 128 * _D, 128 * _D)]

    for u in (0, 1):
        pltpu.async_copy(src_slice(u), src_v.at[u], sin[u])

    def body(g, carry):
        for u in (0, 1):
            i = 2 * g + u
            pltpu.make_async_copy(src_slice(i), src_v.at[u], sin[u]).wait()

            @pl.when(g > 0)
            def _():
                pltpu.make_async_copy(dst_v.at[u], out_slice(i - 2),
                                      sout[u]).wait()

            @plsc.parallel_loop(0, 48, unroll=4)
            def shuf(q):
                t = q // _BANDS
                bb = q % _BANDS
                row48 = (t * 16 + i16) * _D + bb * 8
                for ss in range(8):
                    vals = src_v[u, bb, ss, pl.ds(t * 16, 16)]
                    plsc.store_scatter(dst_v.at[u], [row48 + ss], vals)

            pltpu.async_copy(dst_v.at[u], out_slice(i), sout[u])

            @pl.when(g <= (n_blocks // 2 - 2))
            def _():
                pltpu.async_copy(src_slice(i + 2), src_v.at[u], sin[u])
        return carry

    lax.fori_loop(0, n_blocks // 2, body, 0)
    for u in (0, 1):
        i = n_blocks - 2 + u
        pltpu.make_async_copy(dst_v.at[u], out_slice(i), sout[u]).wait()


@functools.partial(
    pl.kernel,
    out_type=jax.ShapeDtypeStruct((_BANDS * _OT * 1024,), jnp.float32),
    mesh=_mesh,
    scratch_types=[
        pltpu.VMEM((_B // _NW,), jnp.int32),
        pltpu.VMEM((2, 512, _D), jnp.float32),
        pltpu.VMEM((2, 4 * _BANDS * 1024), jnp.float32),
        pltpu.SemaphoreType.DMA,
        pltpu.SemaphoreType.DMA,
        pltpu.SemaphoreType.DMA,
        pltpu.SemaphoreType.DMA,
    ],
    compiler_params=_params,
)
def _gather_kernel(rm_tab, idx_hbm, out_v, idx_v, rows_v, dst_v,
                   sg0, sg1, so0, so1):
    wid = _wid()
    b_per_w = _B // _NW
    n_blocks = b_per_w // 512
    ot0 = wid * (b_per_w // 128)
    sg = (sg0, sg1)
    so = (so0, so1)
    pltpu.sync_copy(idx_hbm.at[pl.ds(wid * b_per_w, b_per_w)], idx_v)
    i16 = lax.iota(jnp.int32, 16)
    flat_idx = []
    for p in range(3):
        dv = p * 16 + i16
        flat_idx.append((dv % 8) * 128 + (dv // 8) * 1024)

    def gather(i, u):
        return pltpu.async_copy(
            rm_tab.at[idx_v.at[pl.ds(i * 512, 512)]], rows_v.at[u], sg[u])

    def out_copies(i, u, fn):
        for sb in range(4):
            ot = ot0 + i * 4 + sb
            for bb in range(_BANDS):
                fn(dst_v.at[u, pl.ds(sb * (_BANDS * 1024) + bb * 1024, 1024)],
                   out_v.at[pl.ds(bb * (_OT * 1024) + ot * 1024, 1024)],
                   so[u])

    for u in (0, 1):
        gather(u, u)

    def body(g, carry):
        for u in (0, 1):
            i = 2 * g + u
            pltpu.make_async_copy(
                rm_tab.at[idx_v.at[pl.ds(i * 512, 512)]],
                rows_v.at[u], sg[u]).wait()

            @pl.when(g > 0)
            def _():
                pltpu.make_async_copy(
                    out_v.at[pl.ds(0, 4 * _BANDS * 1024)],
                    dst_v.at[u], so[u]).wait()

            @plsc.parallel_loop(0, 512, step=4, unroll=4)
            def shuf(j0):
                sb = j0 // 128
                base = sb * (_BANDS * 1024) - sb * 128
                for jj in range(4):
                    j = j0 + jj
                    for p in range(3):
                        vals = rows_v[u, j, pl.ds(p * 16, 16)]
                        plsc.store_scatter(
                            dst_v.at[u], [flat_idx[p] + (base + j)], vals)

            out_copies(i, u, pltpu.async_copy)

            @pl.when(g <= (n_blocks // 2 - 2))
            def _():
                gather(i + 2, u)
        return carry

    lax.fori_loop(0, n_blocks // 2, body, 0)
    for u in (0, 1):
        pltpu.make_async_copy(
            out_v.at[pl.ds(0, 4 * _BANDS * 1024)], dst_v.at[u], so[u]).wait()


def kernel(feat_plane, mask_indices):
    tab_v = feat_plane.reshape(_H, _WT, 128, _BANDS, 8).transpose(0, 3, 1, 4, 2)
    idx = mask_indices.astype(jnp.int32)
    rm = _transpose_kernel(tab_v).reshape(_R, _D)
    out_flat = _gather_kernel(rm, idx)
    return (out_flat.reshape(_BANDS, _OT, 8, 128)
            .transpose(1, 3, 0, 2)
            .reshape(_B, _D))
